# Initial kernel scaffold; baseline (speedup 1.0000x reference)
import jax
import jax.numpy as jnp
from jax.experimental import pallas as pl


def kernel(features, table):
    raise NotImplementedError("write your pallas kernel here")


# speedup vs baseline: 1.5604x; 1.5604x over previous
import functools

import jax
import jax.numpy as jnp
from jax import lax
from jax.experimental import pallas as pl
from jax.experimental.pallas import tpu as pltpu
from jax.experimental.pallas import tpu_sc as plsc

VOCAB = 1000000
EMBED = 32
BATCH = 16384
FIELDS = 26

B = BATCH * FIELDS
IDXW = 128
NROWS = B // IDXW
NC = 2
NS = 16
NW = NC * NS
ROWS_PER_W = NROWS // NW
K = 8
GROUP = K * IDXW
N_GROUPS = ROWS_PER_W // K

_mesh = plsc.VectorSubcoreMesh(core_axis_name="c", subcore_axis_name="s")


@functools.partial(
    pl.kernel,
    mesh=_mesh,
    out_type=jax.ShapeDtypeStruct((B, EMBED), jnp.float32),
    scratch_types=[
        pltpu.VMEM((ROWS_PER_W, IDXW), jnp.int32),
        pltpu.VMEM((GROUP, EMBED), jnp.float32),
        pltpu.SemaphoreType.DMA,
    ],
    compiler_params=pltpu.CompilerParams(use_tc_tiling_on_sc=False),
)
def _sc_gather(idx_hbm, table_hbm, out_hbm, idx_v, rows_v, sem):
    wid = lax.axis_index("s") * NC + lax.axis_index("c")
    row_base = wid * ROWS_PER_W

    pltpu.sync_copy(idx_hbm.at[pl.ds(row_base, ROWS_PER_W)], idx_v)

    def step(g, carry):
        copies = [
            pltpu.async_copy(
                table_hbm.at[idx_v.at[g * K + j]],
                rows_v.at[pl.ds(j * IDXW, IDXW)],
                sem,
            )
            for j in range(K)
        ]
        for c in copies:
            c.wait()
        out_off = (row_base + g * K) * IDXW
        pltpu.sync_copy(rows_v, out_hbm.at[pl.ds(out_off, GROUP)])
        return carry

    lax.fori_loop(0, N_GROUPS, step, 0)


def kernel(features, table):
    idx = features.astype(jnp.int32).reshape(NROWS, IDXW)
    out = _sc_gather(idx, table)
    return out.reshape(BATCH, FIELDS, EMBED)


# speedup vs baseline: 1.5676x; 1.0046x over previous
import functools

import jax
import jax.numpy as jnp
from jax import lax
from jax.experimental import pallas as pl
from jax.experimental.pallas import tpu as pltpu
from jax.experimental.pallas import tpu_sc as plsc

VOCAB = 1000000
EMBED = 32
BATCH = 16384
FIELDS = 26

B = BATCH * FIELDS
NC = 2
NS = 16
NW = NC * NS
B_PER_W = B // NW
CHUNK = 1664
N_GROUPS = B_PER_W // CHUNK

_mesh = plsc.VectorSubcoreMesh(core_axis_name="c", subcore_axis_name="s")


@functools.partial(
    pl.kernel,
    mesh=_mesh,
    out_type=jax.ShapeDtypeStruct((B, EMBED), jnp.float32),
    scratch_types=[
        pltpu.VMEM((B_PER_W,), jnp.int32),
        pltpu.VMEM((CHUNK, EMBED), jnp.float32),
        pltpu.SemaphoreType.DMA,
    ],
    compiler_params=pltpu.CompilerParams(use_tc_tiling_on_sc=False),
)
def _sc_gather(idx_hbm, table_hbm, out_hbm, idx_v, rows_v, sem):
    wid = lax.axis_index("s") * NC + lax.axis_index("c")
    base = wid * B_PER_W

    pltpu.sync_copy(idx_hbm.at[pl.ds(base, B_PER_W)], idx_v)

    def step(g, carry):
        off = g * CHUNK
        pltpu.async_copy(
            table_hbm.at[idx_v.at[pl.ds(off, CHUNK)]], rows_v, sem
        ).wait()
        pltpu.sync_copy(rows_v, out_hbm.at[pl.ds(base + off, CHUNK)])
        return carry

    lax.fori_loop(0, N_GROUPS, step, 0)


def kernel(features, table):
    idx = features.astype(jnp.int32).reshape(B)
    out = _sc_gather(idx, table)
    return out.reshape(BATCH, FIELDS, EMBED)
